# Initial kernel scaffold; baseline (speedup 1.0000x reference)
import functools

import jax
import jax.numpy as jnp
import numpy as np
from jax import lax
from jax.experimental import pallas as pl
from jax.experimental.pallas import tpu as pltpu
from jax.experimental.pallas import tpu_sc as plsc

GRID_SIZE = 5
SPLINE_ORDER = 3
N_NODES = 10000
N_EDGES = 320000
INPUT_DIM = 256
ENC_HID = [50, 32]
DEC_HID = [50, 20]
CONV_HID = [50, 12]
DEC_CLUSTER_N = 15
ALPHA = 0.9
ZDIM = ENC_HID[1] + CONV_HID[1]

NC, NS, LANES = 2, 16, 16
NW = NC * NS
CHUNK = 128
CPW = -(-N_EDGES // (NW * CHUNK))
E_PAD = NW * CPW * CHUNK
N_PAD = N_NODES + 16
ROWS_PER_TILE = N_PAD // NS
F_AGG = 32

_H = np.float32(2.0 / GRID_SIZE)
_PTS = (np.arange(-SPLINE_ORDER, GRID_SIZE + SPLINE_ORDER + 1,
                  dtype=np.float32) * _H - np.float32(1.0))
_NKNOT = len(_PTS)
_RECIP1 = [[float(np.float32(1.0) / (_PTS[j + k] - _PTS[j]))
            for j in range(_NKNOT - 1 - k)] for k in range(SPLINE_ORDER + 1)]
_RECIP2 = [[float(np.float32(1.0) / (_PTS[j + k + 1] - _PTS[j + 1]))
            for j in range(_NKNOT - 1 - k)] for k in range(SPLINE_ORDER + 1)]
_KNOT = [float(p) for p in _PTS]
N_BASES = GRID_SIZE + SPLINE_ORDER


def _bases(xt):
    b = [jnp.where((xt >= _KNOT[j]) & (xt < _KNOT[j + 1]), 1.0, 0.0)
         for j in range(_NKNOT - 1)]
    for k in range(1, SPLINE_ORDER + 1):
        b = [(xt - _KNOT[j]) * _RECIP1[k][j] * b[j]
             + (_KNOT[j + k + 1] - xt) * _RECIP2[k][j] * b[j + 1]
             for j in range(_NKNOT - 1 - k)]
    return b


def _kan_tile(xt, wbase, wspl_ref):
    out = jnp.dot(jax.nn.silu(xt), wbase, preferred_element_type=jnp.float32)
    for j, bj in enumerate(_bases(xt)):
        out = out + jnp.dot(bj, wspl_ref[j], preferred_element_type=jnp.float32)
    return out


def _bn(h, g, b):
    mean = jnp.mean(h, axis=0, keepdims=True)
    var = jnp.mean((h - mean) ** 2, axis=0, keepdims=True)
    return g * (h - mean) * lax.rsqrt(var + 0.001) + b


def _elu(x):
    return jnp.where(x > 0, x, jnp.expm1(x))


_TILE = 1000
_NTILES = N_NODES // _TILE


def _make_deg_kernel():
    mesh = plsc.VectorSubcoreMesh(core_axis_name="c", subcore_axis_name="s")

    @functools.partial(
        pl.kernel,
        out_type=jax.ShapeDtypeStruct((NC * N_PAD, 8), jnp.float32),
        mesh=mesh,
        scratch_types=[
            pltpu.VMEM((CPW, CHUNK), jnp.int32),
            pltpu.VMEM((CHUNK, 8), jnp.float32),
            pltpu.VMEM_SHARED((N_PAD, 8), jnp.float32),
        ],
    )
    def deg_kernel(dsts_hbm, zeros_hbm, ones_hbm, out_hbm, dst_v, ones_v, acc):
        cid = lax.axis_index("c")
        sid = lax.axis_index("s")
        r0 = sid * ROWS_PER_TILE
        pltpu.sync_copy(zeros_hbm.at[pl.ds(r0, ROWS_PER_TILE)],
                        acc.at[pl.ds(r0, ROWS_PER_TILE)])
        pltpu.sync_copy(ones_hbm, ones_v)
        plsc.subcore_barrier()
        base_chunk = (cid * NS + sid) * CPW
        pltpu.sync_copy(dsts_hbm.at[pl.ds(base_chunk, CPW)], dst_v)

        def body(j, carry):
            pltpu.sync_copy(ones_v, acc.at[dst_v.at[j]], add=True)
            return carry

        lax.fori_loop(0, CPW, body, 0)
        plsc.subcore_barrier()
        pltpu.sync_copy(acc.at[pl.ds(r0, ROWS_PER_TILE)],
                        out_hbm.at[pl.ds(cid * N_PAD + r0, ROWS_PER_TILE)])

    return deg_kernel


def _make_agg_kernel():
    mesh = plsc.VectorSubcoreMesh(core_axis_name="c", subcore_axis_name="s")

    @functools.partial(
        pl.kernel,
        out_type=jax.ShapeDtypeStruct((NC * N_PAD, F_AGG), jnp.float32),
        mesh=mesh,
        scratch_types=[
            pltpu.VMEM((CPW, CHUNK), jnp.int32),
            pltpu.VMEM((CPW, CHUNK), jnp.int32),
            pltpu.VMEM((CHUNK, F_AGG), jnp.float32),
            pltpu.VMEM_SHARED((N_PAD, F_AGG), jnp.float32),
            pltpu.SemaphoreType.DMA,
        ],
    )
    def agg_kernel(val_hbm, srcs_hbm, dsts_hbm, zeros_hbm, out_hbm,
                   src_v, dst_v, rows_v, acc, sem):
        cid = lax.axis_index("c")
        sid = lax.axis_index("s")
        r0 = sid * ROWS_PER_TILE
        pltpu.sync_copy(zeros_hbm.at[pl.ds(r0, ROWS_PER_TILE)],
                        acc.at[pl.ds(r0, ROWS_PER_TILE)])
        plsc.subcore_barrier()
        base_chunk = (cid * NS + sid) * CPW
        pltpu.sync_copy(srcs_hbm.at[pl.ds(base_chunk, CPW)], src_v)
        pltpu.sync_copy(dsts_hbm.at[pl.ds(base_chunk, CPW)], dst_v)

        def body(j, carry):
            pltpu.async_copy(val_hbm.at[src_v.at[j]], rows_v, sem).wait()
            pltpu.sync_copy(rows_v, acc.at[dst_v.at[j]], add=True)
            return carry

        lax.fori_loop(0, CPW, body, 0)
        plsc.subcore_barrier()
        pltpu.sync_copy(acc.at[pl.ds(r0, ROWS_PER_TILE)],
                        out_hbm.at[pl.ds(cid * N_PAD + r0, ROWS_PER_TILE)])

    return agg_kernel


_deg_call = _make_deg_kernel()
_agg_call = _make_agg_kernel()


def _enc_body(x_ref, degp_ref, e0bw, e0sw, e0g, e0b, e1bw, e1sw, e1g, e1b,
              feat_ref, scaled_ref, dinv_ref, h0_ref, a0_ref, h1_ref):
    def tile0(i, carry):
        xt = x_ref[pl.ds(i * _TILE, _TILE), :]
        h0_ref[pl.ds(i * _TILE, _TILE), :] = _kan_tile(xt, e0bw[...], e0sw)
        return carry
    lax.fori_loop(0, _NTILES, tile0, 0)
    a0_ref[...] = _elu(_bn(h0_ref[...], e0g[...], e0b[...]))

    def tile1(i, carry):
        at = a0_ref[pl.ds(i * _TILE, _TILE), :]
        h1_ref[pl.ds(i * _TILE, _TILE), :] = _kan_tile(at, e1bw[...], e1sw)
        return carry
    lax.fori_loop(0, _NTILES, tile1, 0)
    feat = _elu(_bn(h1_ref[...], e1g[...], e1b[...]))
    feat_ref[...] = feat

    deg = (degp_ref[pl.ds(0, N_NODES), 0:1]
           + degp_ref[pl.ds(N_PAD, N_NODES), 0:1] + 1.0)
    dinv = lax.rsqrt(deg)
    dinv_ref[...] = dinv
    scaled_ref[pl.ds(0, N_NODES), :] = feat * dinv
    scaled_ref[pl.ds(N_NODES, N_PAD - N_NODES), :] = jnp.zeros(
        (N_PAD - N_NODES, F_AGG), jnp.float32)


def _mid_body(aggp_ref, scaled_ref, dinv_ref, cw, cb, cg, cbeta, mlw,
              ts_ref):
    agg = (aggp_ref[pl.ds(0, N_NODES), :] + aggp_ref[pl.ds(N_PAD, N_NODES), :]
           + scaled_ref[pl.ds(0, N_NODES), :])
    dinv = dinv_ref[...]
    pre = jnp.dot(dinv * agg, cw[...],
                  preferred_element_type=jnp.float32) + cb[...]
    h = jnp.maximum(_bn(pre, cg[...], cbeta[...]), 0.0)
    t = jnp.dot(h, mlw[...], preferred_element_type=jnp.float32)
    ts_ref[pl.ds(0, N_NODES), :] = dinv * t
    ts_ref[pl.ds(N_NODES, N_PAD - N_NODES), :] = jnp.zeros(
        (N_PAD - N_NODES, F_AGG), jnp.float32)


def _dec_body(aggp_ref, ts_ref, dinv_ref, feat_ref, mub, lvb,
              d0bw, d0sw, d0g, d0b, d1bw, d1sw, d1g, d1b, dobw, dosw,
              clus_ref,
              z_ref, mu_ref, lv_ref, defeat_ref, q_ref,
              h0_ref, a0_ref, h1_ref, a1_ref):
    m = (aggp_ref[pl.ds(0, N_NODES), :] + aggp_ref[pl.ds(N_PAD, N_NODES), :]
         + ts_ref[pl.ds(0, N_NODES), :]) * dinv_ref[...]
    mu = m[:, 0:CONV_HID[1]] + mub[...]
    lv = m[:, CONV_HID[1]:2 * CONV_HID[1]] + lvb[...]
    mu_ref[...] = mu
    lv_ref[...] = lv
    z = jnp.concatenate([feat_ref[...], mu], axis=1)
    z_ref[...] = z

    def tile0(i, carry):
        zt = z_ref[pl.ds(i * _TILE, _TILE), :]
        h0_ref[pl.ds(i * _TILE, _TILE), :] = _kan_tile(zt, d0bw[...], d0sw)
        return carry
    lax.fori_loop(0, _NTILES, tile0, 0)
    a0_ref[...] = _elu(_bn(h0_ref[...], d0g[...], d0b[...]))

    def tile1(i, carry):
        at = a0_ref[pl.ds(i * _TILE, _TILE), :]
        h1_ref[pl.ds(i * _TILE, _TILE), :] = _kan_tile(at, d1bw[...], d1sw)
        return carry
    lax.fori_loop(0, _NTILES, tile1, 0)
    a1_ref[...] = _elu(_bn(h1_ref[...], d1g[...], d1b[...]))

    def tile2(i, carry):
        at = a1_ref[pl.ds(i * _TILE, _TILE), :]
        defeat_ref[pl.ds(i * _TILE, _TILE), :] = _kan_tile(at, dobw[...], dosw)
        return carry
    lax.fori_loop(0, _NTILES, tile2, 0)

    clus = clus_ref[...]
    zz = jnp.sum(z * z, axis=1, keepdims=True)
    cc = jnp.sum(clus * clus, axis=1, keepdims=True).T
    zc = jnp.dot(z, clus.T, preferred_element_type=jnp.float32)
    d2 = zz - 2.0 * zc + cc
    t = 1.0 / (1.0 + d2 * (1.0 / ALPHA))
    q = jnp.exp(((ALPHA + 1.0) / 2.0) * jnp.log(t))
    q_ref[...] = q / jnp.sum(q, axis=1, keepdims=True)


def kernel(x, adj, params):
    f32 = jnp.float32
    src = adj[0].astype(jnp.int32)
    dst = adj[1].astype(jnp.int32)
    npad = E_PAD - N_EDGES
    srcs = jnp.concatenate(
        [src, jnp.full((npad,), N_NODES, jnp.int32)]).reshape(NW * CPW, CHUNK)
    dsts = jnp.concatenate(
        [dst, jnp.full((npad,), N_NODES, jnp.int32)]).reshape(NW * CPW, CHUNK)

    zeros8 = jnp.zeros((N_PAD, 8), f32)
    ones8 = jnp.ones((CHUNK, 8), f32)
    zeros32 = jnp.zeros((N_PAD, F_AGG), f32)

    p = params
    e0, e1, d0, d1 = p['enc0'], p['enc1'], p['dec0'], p['dec1']

    def tw(blk):
        return jnp.transpose(blk['spline_w'], (2, 1, 0))

    def row(v):
        return v.reshape(1, -1)

    degp = _deg_call(dsts, zeros8, ones8)

    feat, scaled, dinv = pl.pallas_call(
        _enc_body,
        out_shape=[
            jax.ShapeDtypeStruct((N_NODES, ENC_HID[1]), f32),
            jax.ShapeDtypeStruct((N_PAD, F_AGG), f32),
            jax.ShapeDtypeStruct((N_NODES, 1), f32),
        ],
        scratch_shapes=[
            pltpu.VMEM((N_NODES, ENC_HID[0]), f32),
            pltpu.VMEM((N_NODES, ENC_HID[0]), f32),
            pltpu.VMEM((N_NODES, ENC_HID[1]), f32),
        ],
    )(x, degp, e0['base_w'].T, tw(e0), row(e0['bn_g']), row(e0['bn_b']),
      e1['base_w'].T, tw(e1), row(e1['bn_g']), row(e1['bn_b']))

    aggp1 = _agg_call(scaled, srcs, dsts, zeros32)

    mlw = jnp.concatenate(
        [p['mu_w'].T, p['lv_w'].T,
         jnp.zeros((CONV_HID[0] * 2, F_AGG - 2 * CONV_HID[1]), f32)], axis=1)
    ts = pl.pallas_call(
        _mid_body,
        out_shape=jax.ShapeDtypeStruct((N_PAD, F_AGG), f32),
    )(aggp1, scaled, dinv, p['conv_w'].T, row(p['conv_b']),
      row(p['conv_bn_g']), row(p['conv_bn_b']), mlw)

    aggp2 = _agg_call(ts, srcs, dsts, zeros32)

    do = p['dec_out']
    z, mu, logvar, de_feat, q = pl.pallas_call(
        _dec_body,
        out_shape=[
            jax.ShapeDtypeStruct((N_NODES, ZDIM), f32),
            jax.ShapeDtypeStruct((N_NODES, CONV_HID[1]), f32),
            jax.ShapeDtypeStruct((N_NODES, CONV_HID[1]), f32),
            jax.ShapeDtypeStruct((N_NODES, INPUT_DIM), f32),
            jax.ShapeDtypeStruct((N_NODES, DEC_CLUSTER_N), f32),
        ],
        scratch_shapes=[
            pltpu.VMEM((N_NODES, DEC_HID[0]), f32),
            pltpu.VMEM((N_NODES, DEC_HID[0]), f32),
            pltpu.VMEM((N_NODES, DEC_HID[1]), f32),
            pltpu.VMEM((N_NODES, DEC_HID[1]), f32),
        ],
    )(aggp2, ts, dinv, p['conv_w'].T if False else mlw * 0 if False else ts * 0 if False else feat,
      row(p['mu_b']), row(p['lv_b']),
      d0['base_w'].T, tw(d0), row(d0['bn_g']), row(d0['bn_b']),
      d1['base_w'].T, tw(d1), row(d1['bn_g']), row(d1['bn_b']),
      do['base_w'].T, tw(do), p['cluster'])

    return (z, mu, logvar, de_feat, q, feat, mu)


# speedup vs baseline: 16.7686x; 16.7686x over previous
import functools

import jax
import jax.numpy as jnp
import numpy as np
from jax import lax
from jax.experimental import pallas as pl
from jax.experimental.pallas import tpu as pltpu
from jax.experimental.pallas import tpu_sc as plsc

GRID_SIZE = 5
SPLINE_ORDER = 3
N_NODES = 10000
N_EDGES = 320000
INPUT_DIM = 256
ENC_HID = [50, 32]
DEC_HID = [50, 20]
CONV_HID = [50, 12]
DEC_CLUSTER_N = 15
ALPHA = 0.9
ZDIM = ENC_HID[1] + CONV_HID[1]

NC, NS, LANES = 2, 16, 16
NW = NC * NS
CHUNK = 128
CPW = (-(-N_EDGES // (NW * CHUNK)) + 7) // 8 * 8
E_PAD = NW * CPW * CHUNK
N_PAD = 10112
ROWS_PER_TILE = N_PAD // NS
F_AGG = 32

_H = np.float32(2.0 / GRID_SIZE)
_PTS = (np.arange(-SPLINE_ORDER, GRID_SIZE + SPLINE_ORDER + 1,
                  dtype=np.float32) * _H - np.float32(1.0))
_NKNOT = len(_PTS)
_RECIP1 = {k: [float(np.float32(1.0) / (_PTS[j + k] - _PTS[j]))
               for j in range(_NKNOT - 1 - k)]
           for k in range(1, SPLINE_ORDER + 1)}
_RECIP2 = {k: [float(np.float32(1.0) / (_PTS[j + k + 1] - _PTS[j + 1]))
               for j in range(_NKNOT - 1 - k)]
           for k in range(1, SPLINE_ORDER + 1)}
_KNOT = [float(p) for p in _PTS]
N_BASES = GRID_SIZE + SPLINE_ORDER


def _bases(xt):
    b = [jnp.where((xt >= _KNOT[j]) & (xt < _KNOT[j + 1]), 1.0, 0.0)
         for j in range(_NKNOT - 1)]
    for k in range(1, SPLINE_ORDER + 1):
        b = [(xt - _KNOT[j]) * _RECIP1[k][j] * b[j]
             + (_KNOT[j + k + 1] - xt) * _RECIP2[k][j] * b[j + 1]
             for j in range(_NKNOT - 1 - k)]
    return b


def _kan_tile(xt, wbase, wspl_ref):
    out = jnp.dot(jax.nn.silu(xt), wbase, preferred_element_type=jnp.float32)
    for j, bj in enumerate(_bases(xt)):
        out = out + jnp.dot(bj, wspl_ref[j], preferred_element_type=jnp.float32)
    return out


def _bn(h, g, b):
    mean = jnp.mean(h, axis=0, keepdims=True)
    var = jnp.mean((h - mean) ** 2, axis=0, keepdims=True)
    return g * (h - mean) * lax.rsqrt(var + 0.001) + b


def _elu(x):
    return jnp.where(x > 0, x, jnp.exp(jnp.minimum(x, 0.0)) - 1.0)


_TILE = 400
_NTILES = N_NODES // _TILE


def _make_deg_kernel():
    mesh = plsc.VectorSubcoreMesh(core_axis_name="c", subcore_axis_name="s",
                                   num_cores=NC, num_subcores=NS)

    @functools.partial(
        pl.kernel,
        out_type=jax.ShapeDtypeStruct((NC * N_PAD, 8), jnp.float32),
        mesh=mesh,
        scratch_types=[
            pltpu.VMEM((CPW, CHUNK), jnp.int32),
            pltpu.VMEM((CHUNK, 8), jnp.float32),
            pltpu.VMEM_SHARED((N_PAD, 8), jnp.float32),
        ],
        compiler_params=pltpu.CompilerParams(use_tc_tiling_on_sc=False),
    )
    def deg_kernel(dsts_hbm, zeros_hbm, ones_hbm, out_hbm, dst_v, ones_v, acc):
        cid = lax.axis_index("c")
        sid = lax.axis_index("s")
        r0 = sid * ROWS_PER_TILE
        pltpu.sync_copy(zeros_hbm.at[pl.ds(r0, ROWS_PER_TILE)],
                        acc.at[pl.ds(r0, ROWS_PER_TILE)])
        pltpu.sync_copy(ones_hbm, ones_v)
        plsc.subcore_barrier()
        base_chunk = (cid * NS + sid) * CPW
        pltpu.sync_copy(dsts_hbm.at[pl.ds(base_chunk, CPW)], dst_v)

        def body(j, carry):
            pltpu.sync_copy(ones_v, acc.at[dst_v.at[j]], add=True)
            return carry

        lax.fori_loop(0, CPW, body, 0)
        plsc.subcore_barrier()
        pltpu.sync_copy(acc.at[pl.ds(r0, ROWS_PER_TILE)],
                        out_hbm.at[pl.ds(cid * N_PAD + r0, ROWS_PER_TILE)])

    return deg_kernel


def _make_agg_kernel():
    mesh = plsc.VectorSubcoreMesh(core_axis_name="c", subcore_axis_name="s",
                                   num_cores=NC, num_subcores=NS)

    @functools.partial(
        pl.kernel,
        out_type=jax.ShapeDtypeStruct((NC * N_PAD, F_AGG), jnp.float32),
        mesh=mesh,
        scratch_types=[
            pltpu.VMEM((CPW, CHUNK), jnp.int32),
            pltpu.VMEM((CPW, CHUNK), jnp.int32),
            pltpu.VMEM((CHUNK, F_AGG), jnp.float32),
            pltpu.VMEM_SHARED((N_PAD, F_AGG), jnp.float32),
            pltpu.SemaphoreType.DMA,
        ],
        compiler_params=pltpu.CompilerParams(use_tc_tiling_on_sc=False),
    )
    def agg_kernel(val_hbm, srcs_hbm, dsts_hbm, zeros_hbm, out_hbm,
                   src_v, dst_v, rows_v, acc, sem):
        cid = lax.axis_index("c")
        sid = lax.axis_index("s")
        r0 = sid * ROWS_PER_TILE
        pltpu.sync_copy(zeros_hbm.at[pl.ds(r0, ROWS_PER_TILE)],
                        acc.at[pl.ds(r0, ROWS_PER_TILE)])
        plsc.subcore_barrier()
        base_chunk = (cid * NS + sid) * CPW
        pltpu.sync_copy(srcs_hbm.at[pl.ds(base_chunk, CPW)], src_v)
        pltpu.sync_copy(dsts_hbm.at[pl.ds(base_chunk, CPW)], dst_v)

        def body(j, carry):
            pltpu.async_copy(val_hbm.at[src_v.at[j]], rows_v, sem).wait()
            pltpu.sync_copy(rows_v, acc.at[dst_v.at[j]], add=True)
            return carry

        lax.fori_loop(0, CPW, body, 0)
        plsc.subcore_barrier()
        pltpu.sync_copy(acc.at[pl.ds(r0, ROWS_PER_TILE)],
                        out_hbm.at[pl.ds(cid * N_PAD + r0, ROWS_PER_TILE)])

    return agg_kernel


_SC_CACHE = {}


def _deg_call(*args):
    if 'deg' not in _SC_CACHE:
        _SC_CACHE['deg'] = _make_deg_kernel()
    return _SC_CACHE['deg'](*args)


def _agg_call(*args):
    if 'agg' not in _SC_CACHE:
        _SC_CACHE['agg'] = _make_agg_kernel()
    return _SC_CACHE['agg'](*args)


def _enc0_body(x_ref, e0bw, e0sw, e0g, e0b, a0_ref):
    def tile(i, carry):
        xt = x_ref[pl.ds(i * _TILE, _TILE), :]
        a0_ref[pl.ds(i * _TILE, _TILE), :] = _kan_tile(xt, e0bw[...], e0sw)
        return carry
    lax.fori_loop(0, _NTILES, tile, 0)
    a0_ref[...] = _elu(_bn(a0_ref[...], e0g[...], e0b[...]))


def _enc1_body(a0_ref, degp_ref, e1bw, e1sw, e1g, e1b,
               feat_ref, scaled_ref, dinv_ref):
    def tile(i, carry):
        at = a0_ref[pl.ds(i * _TILE, _TILE), :]
        feat_ref[pl.ds(i * _TILE, _TILE), :] = _kan_tile(at, e1bw[...], e1sw)
        return carry
    lax.fori_loop(0, _NTILES, tile, 0)
    feat = _elu(_bn(feat_ref[...], e1g[...], e1b[...]))
    feat_ref[...] = feat
    deg = (degp_ref[pl.ds(0, N_NODES), 0:1]
           + degp_ref[pl.ds(N_PAD, N_NODES), 0:1] + 1.0)
    dinv = lax.rsqrt(deg)
    dinv_ref[...] = dinv
    scaled_ref[pl.ds(0, N_NODES), :] = feat * dinv
    scaled_ref[pl.ds(N_NODES, N_PAD - N_NODES), :] = jnp.zeros(
        (N_PAD - N_NODES, F_AGG), jnp.float32)


def _mid_body(aggp_ref, scaled_ref, dinv_ref, cw, cb, cg, cbeta, mlw,
              ts_ref):
    agg = (aggp_ref[pl.ds(0, N_NODES), :] + aggp_ref[pl.ds(N_PAD, N_NODES), :]
           + scaled_ref[pl.ds(0, N_NODES), :])
    dinv = dinv_ref[...]
    pre = jnp.dot(dinv * agg, cw[...],
                  preferred_element_type=jnp.float32) + cb[...]
    h = jnp.maximum(_bn(pre, cg[...], cbeta[...]), 0.0)
    t = jnp.dot(h, mlw[...], preferred_element_type=jnp.float32)
    ts_ref[pl.ds(0, N_NODES), :] = jnp.concatenate(
        [dinv * t, dinv,
         jnp.zeros((N_NODES, F_AGG - 2 * CONV_HID[1] - 1), jnp.float32)],
        axis=1)
    ts_ref[pl.ds(N_NODES, N_PAD - N_NODES), :] = jnp.zeros(
        (N_PAD - N_NODES, F_AGG), jnp.float32)


def _dec01_body(aggp_ref, ts_ref, feat_ref, mublv,
                d0bw, d0sw, d0g, d0b, d1bw, d1sw, d1g, d1b,
                z_ref, mv_ref, a1_ref, d0_ref):
    tsv = ts_ref[pl.ds(0, N_NODES), :]
    dinv = tsv[:, 2 * CONV_HID[1]:2 * CONV_HID[1] + 1]
    m = ((aggp_ref[pl.ds(0, N_NODES), 0:2 * CONV_HID[1]]
          + aggp_ref[pl.ds(N_PAD, N_NODES), 0:2 * CONV_HID[1]]
          + tsv[:, 0:2 * CONV_HID[1]]) * dinv)
    mv = m + mublv[...]
    mv_ref[...] = mv
    z_ref[...] = jnp.concatenate([feat_ref[...], mv[:, 0:CONV_HID[1]]], axis=1)

    def tile0(i, carry):
        zt = z_ref[pl.ds(i * _TILE, _TILE), :]
        d0_ref[pl.ds(i * _TILE, _TILE), :] = _kan_tile(zt, d0bw[...], d0sw)
        return carry
    lax.fori_loop(0, _NTILES, tile0, 0)
    d0_ref[...] = _elu(_bn(d0_ref[...], d0g[...], d0b[...]))

    def tile1(i, carry):
        at = d0_ref[pl.ds(i * _TILE, _TILE), :]
        a1_ref[pl.ds(i * _TILE, _TILE), :] = _kan_tile(at, d1bw[...], d1sw)
        return carry
    lax.fori_loop(0, _NTILES, tile1, 0)
    a1_ref[...] = _elu(_bn(a1_ref[...], d1g[...], d1b[...]))


def _decout_body(a1_ref, z_ref, dobw, dosw, clus_ref, defeat_ref, q_ref):
    def tile(i, carry):
        at = a1_ref[pl.ds(i * _TILE, _TILE), :]
        defeat_ref[pl.ds(i * _TILE, _TILE), :] = _kan_tile(at, dobw[...], dosw)
        return carry
    lax.fori_loop(0, _NTILES, tile, 0)

    z = z_ref[...]
    clus = clus_ref[...]
    zz = jnp.sum(z * z, axis=1, keepdims=True)
    cc = jnp.sum(clus * clus, axis=1, keepdims=True).T
    zc = jnp.dot(z, clus.T, preferred_element_type=jnp.float32)
    d2 = zz - 2.0 * zc + cc
    t = 1.0 / (1.0 + d2 * (1.0 / ALPHA))
    q = jnp.exp(((ALPHA + 1.0) / 2.0) * jnp.log(t))
    q_ref[...] = q / jnp.sum(q, axis=1, keepdims=True)


def kernel(x, adj, params):
    f32 = jnp.float32
    src = adj[0].astype(jnp.int32)
    dst = adj[1].astype(jnp.int32)
    npad = E_PAD - N_EDGES
    srcs = jnp.concatenate(
        [src, jnp.full((npad,), N_NODES, jnp.int32)]).reshape(NW * CPW, CHUNK)
    dsts = jnp.concatenate(
        [dst, jnp.full((npad,), N_NODES, jnp.int32)]).reshape(NW * CPW, CHUNK)

    zeros8 = jnp.zeros((N_PAD, 8), f32)
    ones8 = jnp.ones((CHUNK, 8), f32)
    zeros32 = jnp.zeros((N_PAD, F_AGG), f32)

    p = params
    e0, e1, d0, d1 = p['enc0'], p['enc1'], p['dec0'], p['dec1']

    def tw(blk):
        return jnp.transpose(blk['spline_w'], (2, 1, 0))

    def row(v):
        return v.reshape(1, -1)

    degp = _deg_call(dsts, zeros8, ones8)

    a0 = pl.pallas_call(
        _enc0_body,
        out_shape=jax.ShapeDtypeStruct((N_NODES, ENC_HID[0]), f32),
    )(x, e0['base_w'].T, tw(e0), row(e0['bn_g']), row(e0['bn_b']))

    feat, scaled, dinv = pl.pallas_call(
        _enc1_body,
        out_shape=[
            jax.ShapeDtypeStruct((N_NODES, ENC_HID[1]), f32),
            jax.ShapeDtypeStruct((N_PAD, F_AGG), f32),
            jax.ShapeDtypeStruct((N_NODES, 1), f32),
        ],
    )(a0, degp, e1['base_w'].T, tw(e1), row(e1['bn_g']), row(e1['bn_b']))

    aggp1 = _agg_call(scaled, srcs, dsts, zeros32)

    mlw = jnp.concatenate([p['mu_w'].T, p['lv_w'].T], axis=1)
    ts = pl.pallas_call(
        _mid_body,
        out_shape=jax.ShapeDtypeStruct((N_PAD, F_AGG), f32),
    )(aggp1, scaled, dinv, p['conv_w'].T, row(p['conv_b']),
      row(p['conv_bn_g']), row(p['conv_bn_b']), mlw)

    aggp2 = _agg_call(ts, srcs, dsts, zeros32)

    mublv = row(jnp.concatenate([p['mu_b'], p['lv_b']]))
    z, mv, a1 = pl.pallas_call(
        _dec01_body,
        out_shape=[
            jax.ShapeDtypeStruct((N_NODES, ZDIM), f32),
            jax.ShapeDtypeStruct((N_NODES, 2 * CONV_HID[1]), f32),
            jax.ShapeDtypeStruct((N_NODES, DEC_HID[1]), f32),
        ],
        scratch_shapes=[pltpu.VMEM((N_NODES, DEC_HID[0]), f32)],
    )(aggp2, ts, feat, mublv,
      d0['base_w'].T, tw(d0), row(d0['bn_g']), row(d0['bn_b']),
      d1['base_w'].T, tw(d1), row(d1['bn_g']), row(d1['bn_b']))

    do = p['dec_out']
    de_feat, q = pl.pallas_call(
        _decout_body,
        out_shape=[
            jax.ShapeDtypeStruct((N_NODES, INPUT_DIM), f32),
            jax.ShapeDtypeStruct((N_NODES, DEC_CLUSTER_N), f32),
        ],
    )(a1, z, do['base_w'].T, tw(do), p['cluster'])

    mu = mv[:, 0:CONV_HID[1]]
    logvar = mv[:, CONV_HID[1]:2 * CONV_HID[1]]
    return (z, mu, logvar, de_feat, q, feat, mu)


# speedup vs baseline: 18.7807x; 1.1200x over previous
import functools

import jax
import jax.numpy as jnp
import numpy as np
from jax import lax
from jax.experimental import pallas as pl
from jax.experimental.pallas import tpu as pltpu
from jax.experimental.pallas import tpu_sc as plsc

GRID_SIZE = 5
SPLINE_ORDER = 3
N_NODES = 10000
N_EDGES = 320000
INPUT_DIM = 256
ENC_HID = [50, 32]
DEC_HID = [50, 20]
CONV_HID = [50, 12]
DEC_CLUSTER_N = 15
ALPHA = 0.9
ZDIM = ENC_HID[1] + CONV_HID[1]

NC, NS, LANES = 2, 16, 16
NW = NC * NS
CHUNK = 128
CPW = (-(-N_EDGES // (NW * CHUNK)) + 7) // 8 * 8
E_PAD = NW * CPW * CHUNK
N_PAD = 10112
ROWS_PER_TILE = N_PAD // NS
F_AGG = 32

_H = np.float32(2.0 / GRID_SIZE)
_PTS = (np.arange(-SPLINE_ORDER, GRID_SIZE + SPLINE_ORDER + 1,
                  dtype=np.float32) * _H - np.float32(1.0))
_NKNOT = len(_PTS)
_RECIP1 = {k: [float(np.float32(1.0) / (_PTS[j + k] - _PTS[j]))
               for j in range(_NKNOT - 1 - k)]
           for k in range(1, SPLINE_ORDER + 1)}
_RECIP2 = {k: [float(np.float32(1.0) / (_PTS[j + k + 1] - _PTS[j + 1]))
               for j in range(_NKNOT - 1 - k)]
           for k in range(1, SPLINE_ORDER + 1)}
_KNOT = [float(p) for p in _PTS]
N_BASES = GRID_SIZE + SPLINE_ORDER


def _bases(xt):
    b = [jnp.where((xt >= _KNOT[j]) & (xt < _KNOT[j + 1]), 1.0, 0.0)
         for j in range(_NKNOT - 1)]
    for k in range(1, SPLINE_ORDER + 1):
        b = [(xt - _KNOT[j]) * _RECIP1[k][j] * b[j]
             + (_KNOT[j + k + 1] - xt) * _RECIP2[k][j] * b[j + 1]
             for j in range(_NKNOT - 1 - k)]
    return b


def _kan_tile(xt, wbase, wspl_ref):
    out = jnp.dot(jax.nn.silu(xt), wbase, preferred_element_type=jnp.float32)
    for j, bj in enumerate(_bases(xt)):
        out = out + jnp.dot(bj, wspl_ref[j], preferred_element_type=jnp.float32)
    return out


def _bn(h, g, b):
    mean = jnp.mean(h, axis=0, keepdims=True)
    var = jnp.mean((h - mean) ** 2, axis=0, keepdims=True)
    return g * (h - mean) * lax.rsqrt(var + 0.001) + b


def _elu(x):
    return jnp.where(x > 0, x, jnp.exp(jnp.minimum(x, 0.0)) - 1.0)


_TILE = 400
_NTILES = N_NODES // _TILE


def _make_deg_kernel():
    mesh = plsc.VectorSubcoreMesh(core_axis_name="c", subcore_axis_name="s",
                                   num_cores=NC, num_subcores=NS)

    @functools.partial(
        pl.kernel,
        out_type=jax.ShapeDtypeStruct((NC * N_PAD, 8), jnp.float32),
        mesh=mesh,
        scratch_types=[
            pltpu.VMEM((CPW, CHUNK), jnp.int32),
            pltpu.VMEM((CHUNK, 8), jnp.float32),
            pltpu.VMEM_SHARED((N_PAD, 8), jnp.float32),
        ],
        compiler_params=pltpu.CompilerParams(use_tc_tiling_on_sc=False),
    )
    def deg_kernel(dsts_hbm, zeros_hbm, ones_hbm, out_hbm, dst_v, ones_v, acc):
        cid = lax.axis_index("c")
        sid = lax.axis_index("s")
        r0 = sid * ROWS_PER_TILE
        pltpu.sync_copy(zeros_hbm.at[pl.ds(r0, ROWS_PER_TILE)],
                        acc.at[pl.ds(r0, ROWS_PER_TILE)])
        pltpu.sync_copy(ones_hbm, ones_v)
        plsc.subcore_barrier()
        base_chunk = (cid * NS + sid) * CPW
        pltpu.sync_copy(dsts_hbm.at[pl.ds(base_chunk, CPW)], dst_v)

        def body(j, carry):
            pltpu.sync_copy(ones_v, acc.at[dst_v.at[j]], add=True)
            return carry

        lax.fori_loop(0, CPW, body, 0)
        plsc.subcore_barrier()
        pltpu.sync_copy(acc.at[pl.ds(r0, ROWS_PER_TILE)],
                        out_hbm.at[pl.ds(cid * N_PAD + r0, ROWS_PER_TILE)])

    return deg_kernel


def _make_agg_kernel():
    mesh = plsc.VectorSubcoreMesh(core_axis_name="c", subcore_axis_name="s",
                                   num_cores=NC, num_subcores=NS)

    @functools.partial(
        pl.kernel,
        out_type=jax.ShapeDtypeStruct((NC * N_PAD, F_AGG), jnp.float32),
        mesh=mesh,
        scratch_types=[
            pltpu.VMEM((CPW, CHUNK), jnp.int32),
            pltpu.VMEM((CPW, CHUNK), jnp.int32),
            pltpu.VMEM((CHUNK, F_AGG), jnp.float32),
            pltpu.VMEM((CHUNK, F_AGG), jnp.float32),
            pltpu.VMEM_SHARED((N_PAD, F_AGG), jnp.float32),
            pltpu.SemaphoreType.DMA,
            pltpu.SemaphoreType.DMA,
        ],
        compiler_params=pltpu.CompilerParams(use_tc_tiling_on_sc=False),
    )
    def agg_kernel(val_hbm, srcs_hbm, dsts_hbm, zeros_hbm, out_hbm,
                   src_v, dst_v, rows0_v, rows1_v, acc, sem0, sem1):
        cid = lax.axis_index("c")
        sid = lax.axis_index("s")
        r0 = sid * ROWS_PER_TILE
        pltpu.sync_copy(zeros_hbm.at[pl.ds(r0, ROWS_PER_TILE)],
                        acc.at[pl.ds(r0, ROWS_PER_TILE)])
        plsc.subcore_barrier()
        base_chunk = (cid * NS + sid) * CPW
        pltpu.sync_copy(srcs_hbm.at[pl.ds(base_chunk, CPW)], src_v)
        pltpu.sync_copy(dsts_hbm.at[pl.ds(base_chunk, CPW)], dst_v)

        pltpu.async_copy(val_hbm.at[src_v.at[0]], rows0_v, sem0)

        def body(i, carry):
            g = i * 2
            pltpu.async_copy(val_hbm.at[src_v.at[g + 1]], rows1_v, sem1)
            pltpu.make_async_copy(val_hbm.at[src_v.at[g]], rows0_v, sem0).wait()
            pltpu.sync_copy(rows0_v, acc.at[dst_v.at[g]], add=True)

            @pl.when(g + 2 < CPW)
            def _():
                pltpu.async_copy(val_hbm.at[src_v.at[g + 2]], rows0_v, sem0)

            pltpu.make_async_copy(val_hbm.at[src_v.at[g + 1]], rows1_v,
                                  sem1).wait()
            pltpu.sync_copy(rows1_v, acc.at[dst_v.at[g + 1]], add=True)
            return carry

        lax.fori_loop(0, CPW // 2, body, 0)
        plsc.subcore_barrier()
        pltpu.sync_copy(acc.at[pl.ds(r0, ROWS_PER_TILE)],
                        out_hbm.at[pl.ds(cid * N_PAD + r0, ROWS_PER_TILE)])

    return agg_kernel


_SC_CACHE = {}


def _deg_call(*args):
    if 'deg' not in _SC_CACHE:
        _SC_CACHE['deg'] = _make_deg_kernel()
    return _SC_CACHE['deg'](*args)


def _agg_call(*args):
    if 'agg' not in _SC_CACHE:
        _SC_CACHE['agg'] = _make_agg_kernel()
    return _SC_CACHE['agg'](*args)


def _enc0_body(x_ref, e0bw, e0sw, e0g, e0b, a0_ref):
    def tile(i, carry):
        xt = x_ref[pl.ds(i * _TILE, _TILE), :]
        a0_ref[pl.ds(i * _TILE, _TILE), :] = _kan_tile(xt, e0bw[...], e0sw)
        return carry
    lax.fori_loop(0, _NTILES, tile, 0)
    a0_ref[...] = _elu(_bn(a0_ref[...], e0g[...], e0b[...]))


def _enc1_body(a0_ref, degp_ref, e1bw, e1sw, e1g, e1b,
               feat_ref, scaled_ref, dinv_ref):
    def tile(i, carry):
        at = a0_ref[pl.ds(i * _TILE, _TILE), :]
        feat_ref[pl.ds(i * _TILE, _TILE), :] = _kan_tile(at, e1bw[...], e1sw)
        return carry
    lax.fori_loop(0, _NTILES, tile, 0)
    feat = _elu(_bn(feat_ref[...], e1g[...], e1b[...]))
    feat_ref[...] = feat
    deg = (degp_ref[pl.ds(0, N_NODES), 0:1]
           + degp_ref[pl.ds(N_PAD, N_NODES), 0:1] + 1.0)
    dinv = lax.rsqrt(deg)
    dinv_ref[...] = dinv
    scaled_ref[pl.ds(0, N_NODES), :] = feat * dinv
    scaled_ref[pl.ds(N_NODES, N_PAD - N_NODES), :] = jnp.zeros(
        (N_PAD - N_NODES, F_AGG), jnp.float32)


def _mid_body(aggp_ref, scaled_ref, dinv_ref, cw, cb, cg, cbeta, mlw,
              ts_ref):
    agg = (aggp_ref[pl.ds(0, N_NODES), :] + aggp_ref[pl.ds(N_PAD, N_NODES), :]
           + scaled_ref[pl.ds(0, N_NODES), :])
    dinv = dinv_ref[...]
    pre = jnp.dot(dinv * agg, cw[...],
                  preferred_element_type=jnp.float32) + cb[...]
    h = jnp.maximum(_bn(pre, cg[...], cbeta[...]), 0.0)
    t = jnp.dot(h, mlw[...], preferred_element_type=jnp.float32)
    ts_ref[pl.ds(0, N_NODES), :] = jnp.concatenate(
        [dinv * t, dinv,
         jnp.zeros((N_NODES, F_AGG - 2 * CONV_HID[1] - 1), jnp.float32)],
        axis=1)
    ts_ref[pl.ds(N_NODES, N_PAD - N_NODES), :] = jnp.zeros(
        (N_PAD - N_NODES, F_AGG), jnp.float32)


def _dec01_body(aggp_ref, ts_ref, feat_ref, mublv,
                d0bw, d0sw, d0g, d0b, d1bw, d1sw, d1g, d1b,
                z_ref, mv_ref, a1_ref, d0_ref):
    tsv = ts_ref[pl.ds(0, N_NODES), :]
    dinv = tsv[:, 2 * CONV_HID[1]:2 * CONV_HID[1] + 1]
    m = ((aggp_ref[pl.ds(0, N_NODES), 0:2 * CONV_HID[1]]
          + aggp_ref[pl.ds(N_PAD, N_NODES), 0:2 * CONV_HID[1]]
          + tsv[:, 0:2 * CONV_HID[1]]) * dinv)
    mv = m + mublv[...]
    mv_ref[...] = mv
    z_ref[...] = jnp.concatenate([feat_ref[...], mv[:, 0:CONV_HID[1]]], axis=1)

    def tile0(i, carry):
        zt = z_ref[pl.ds(i * _TILE, _TILE), :]
        d0_ref[pl.ds(i * _TILE, _TILE), :] = _kan_tile(zt, d0bw[...], d0sw)
        return carry
    lax.fori_loop(0, _NTILES, tile0, 0)
    d0_ref[...] = _elu(_bn(d0_ref[...], d0g[...], d0b[...]))

    def tile1(i, carry):
        at = d0_ref[pl.ds(i * _TILE, _TILE), :]
        a1_ref[pl.ds(i * _TILE, _TILE), :] = _kan_tile(at, d1bw[...], d1sw)
        return carry
    lax.fori_loop(0, _NTILES, tile1, 0)
    a1_ref[...] = _elu(_bn(a1_ref[...], d1g[...], d1b[...]))


def _decout_body(a1_ref, z_ref, dobw, dosw, clus_ref, defeat_ref, q_ref):
    def tile(i, carry):
        at = a1_ref[pl.ds(i * _TILE, _TILE), :]
        defeat_ref[pl.ds(i * _TILE, _TILE), :] = _kan_tile(at, dobw[...], dosw)
        return carry
    lax.fori_loop(0, _NTILES, tile, 0)

    z = z_ref[...]
    clus = clus_ref[...]
    zz = jnp.sum(z * z, axis=1, keepdims=True)
    cc = jnp.sum(clus * clus, axis=1, keepdims=True).T
    zc = jnp.dot(z, clus.T, preferred_element_type=jnp.float32)
    d2 = zz - 2.0 * zc + cc
    t = 1.0 / (1.0 + d2 * (1.0 / ALPHA))
    q = jnp.exp(((ALPHA + 1.0) / 2.0) * jnp.log(t))
    q_ref[...] = q / jnp.sum(q, axis=1, keepdims=True)


def kernel(x, adj, params):
    f32 = jnp.float32
    src = adj[0].astype(jnp.int32)
    dst = adj[1].astype(jnp.int32)
    npad = E_PAD - N_EDGES
    srcs = jnp.concatenate(
        [src, jnp.full((npad,), N_NODES, jnp.int32)]).reshape(NW * CPW, CHUNK)
    dsts = jnp.concatenate(
        [dst, jnp.full((npad,), N_NODES, jnp.int32)]).reshape(NW * CPW, CHUNK)

    zeros8 = jnp.zeros((N_PAD, 8), f32)
    ones8 = jnp.ones((CHUNK, 8), f32)
    zeros32 = jnp.zeros((N_PAD, F_AGG), f32)

    p = params
    e0, e1, d0, d1 = p['enc0'], p['enc1'], p['dec0'], p['dec1']

    def tw(blk):
        return jnp.transpose(blk['spline_w'], (2, 1, 0))

    def row(v):
        return v.reshape(1, -1)

    degp = _deg_call(dsts, zeros8, ones8)

    a0 = pl.pallas_call(
        _enc0_body,
        out_shape=jax.ShapeDtypeStruct((N_NODES, ENC_HID[0]), f32),
    )(x, e0['base_w'].T, tw(e0), row(e0['bn_g']), row(e0['bn_b']))

    feat, scaled, dinv = pl.pallas_call(
        _enc1_body,
        out_shape=[
            jax.ShapeDtypeStruct((N_NODES, ENC_HID[1]), f32),
            jax.ShapeDtypeStruct((N_PAD, F_AGG), f32),
            jax.ShapeDtypeStruct((N_NODES, 1), f32),
        ],
    )(a0, degp, e1['base_w'].T, tw(e1), row(e1['bn_g']), row(e1['bn_b']))

    aggp1 = _agg_call(scaled, srcs, dsts, zeros32)

    mlw = jnp.concatenate([p['mu_w'].T, p['lv_w'].T], axis=1)
    ts = pl.pallas_call(
        _mid_body,
        out_shape=jax.ShapeDtypeStruct((N_PAD, F_AGG), f32),
    )(aggp1, scaled, dinv, p['conv_w'].T, row(p['conv_b']),
      row(p['conv_bn_g']), row(p['conv_bn_b']), mlw)

    aggp2 = _agg_call(ts, srcs, dsts, zeros32)

    mublv = row(jnp.concatenate([p['mu_b'], p['lv_b']]))
    z, mv, a1 = pl.pallas_call(
        _dec01_body,
        out_shape=[
            jax.ShapeDtypeStruct((N_NODES, ZDIM), f32),
            jax.ShapeDtypeStruct((N_NODES, 2 * CONV_HID[1]), f32),
            jax.ShapeDtypeStruct((N_NODES, DEC_HID[1]), f32),
        ],
        scratch_shapes=[pltpu.VMEM((N_NODES, DEC_HID[0]), f32)],
    )(aggp2, ts, feat, mublv,
      d0['base_w'].T, tw(d0), row(d0['bn_g']), row(d0['bn_b']),
      d1['base_w'].T, tw(d1), row(d1['bn_g']), row(d1['bn_b']))

    do = p['dec_out']
    de_feat, q = pl.pallas_call(
        _decout_body,
        out_shape=[
            jax.ShapeDtypeStruct((N_NODES, INPUT_DIM), f32),
            jax.ShapeDtypeStruct((N_NODES, DEC_CLUSTER_N), f32),
        ],
    )(a1, z, do['base_w'].T, tw(do), p['cluster'])

    mu = mv[:, 0:CONV_HID[1]]
    logvar = mv[:, CONV_HID[1]:2 * CONV_HID[1]]
    return (z, mu, logvar, de_feat, q, feat, mu)


# speedup vs baseline: 18.8480x; 1.0036x over previous
import functools

import jax
import jax.numpy as jnp
import numpy as np
from jax import lax
from jax.experimental import pallas as pl
from jax.experimental.pallas import tpu as pltpu
from jax.experimental.pallas import tpu_sc as plsc

GRID_SIZE = 5
SPLINE_ORDER = 3
N_NODES = 10000
N_EDGES = 320000
INPUT_DIM = 256
ENC_HID = [50, 32]
DEC_HID = [50, 20]
CONV_HID = [50, 12]
DEC_CLUSTER_N = 15
ALPHA = 0.9
ZDIM = ENC_HID[1] + CONV_HID[1]

NC, NS, LANES = 2, 16, 16
NW = NC * NS
CHUNK = 128
CPW = (-(-N_EDGES // (NW * CHUNK)) + 7) // 8 * 8
E_PAD = NW * CPW * CHUNK
N_PAD = 10112
ROWS_PER_TILE = N_PAD // NS
F_AGG = 32
_NBUF = 4

_H = np.float32(2.0 / GRID_SIZE)
_PTS = (np.arange(-SPLINE_ORDER, GRID_SIZE + SPLINE_ORDER + 1,
                  dtype=np.float32) * _H - np.float32(1.0))
_NKNOT = len(_PTS)
_RECIP1 = {k: [float(np.float32(1.0) / (_PTS[j + k] - _PTS[j]))
               for j in range(_NKNOT - 1 - k)]
           for k in range(1, SPLINE_ORDER + 1)}
_RECIP2 = {k: [float(np.float32(1.0) / (_PTS[j + k + 1] - _PTS[j + 1]))
               for j in range(_NKNOT - 1 - k)]
           for k in range(1, SPLINE_ORDER + 1)}
_KNOT = [float(p) for p in _PTS]
N_BASES = GRID_SIZE + SPLINE_ORDER


def _bases(xt):
    b = [jnp.where((xt >= _KNOT[j]) & (xt < _KNOT[j + 1]), 1.0, 0.0)
         for j in range(_NKNOT - 1)]
    for k in range(1, SPLINE_ORDER + 1):
        b = [(xt - _KNOT[j]) * _RECIP1[k][j] * b[j]
             + (_KNOT[j + k + 1] - xt) * _RECIP2[k][j] * b[j + 1]
             for j in range(_NKNOT - 1 - k)]
    return b


def _kan_tile(xt, wbase, wspl_ref):
    out = jnp.dot(jax.nn.silu(xt), wbase, preferred_element_type=jnp.float32)
    for j, bj in enumerate(_bases(xt)):
        out = out + jnp.dot(bj, wspl_ref[j], preferred_element_type=jnp.float32)
    return out


def _bn(h, g, b):
    mean = jnp.mean(h, axis=0, keepdims=True)
    var = jnp.mean((h - mean) ** 2, axis=0, keepdims=True)
    return g * (h - mean) * lax.rsqrt(var + 0.001) + b


def _elu(x):
    return jnp.where(x > 0, x, jnp.exp(jnp.minimum(x, 0.0)) - 1.0)


_TILE = 400
_NTILES = N_NODES // _TILE


def _make_deg_kernel():
    mesh = plsc.VectorSubcoreMesh(core_axis_name="c", subcore_axis_name="s",
                                   num_cores=NC, num_subcores=NS)

    @functools.partial(
        pl.kernel,
        out_type=jax.ShapeDtypeStruct((NC * N_PAD, 8), jnp.float32),
        mesh=mesh,
        scratch_types=[
            pltpu.VMEM((CPW, CHUNK), jnp.int32),
            pltpu.VMEM((CHUNK, 8), jnp.float32),
            pltpu.VMEM_SHARED((N_PAD, 8), jnp.float32),
        ],
        compiler_params=pltpu.CompilerParams(use_tc_tiling_on_sc=False),
    )
    def deg_kernel(dsts_hbm, zeros_hbm, ones_hbm, out_hbm, dst_v, ones_v, acc):
        cid = lax.axis_index("c")
        sid = lax.axis_index("s")
        r0 = sid * ROWS_PER_TILE
        pltpu.sync_copy(zeros_hbm.at[pl.ds(r0, ROWS_PER_TILE)],
                        acc.at[pl.ds(r0, ROWS_PER_TILE)])
        pltpu.sync_copy(ones_hbm, ones_v)
        plsc.subcore_barrier()
        base_chunk = (cid * NS + sid) * CPW
        pltpu.sync_copy(dsts_hbm.at[pl.ds(base_chunk, CPW)], dst_v)

        def body(j, carry):
            pltpu.sync_copy(ones_v, acc.at[dst_v.at[j]], add=True)
            return carry

        lax.fori_loop(0, CPW, body, 0)
        plsc.subcore_barrier()
        pltpu.sync_copy(acc.at[pl.ds(r0, ROWS_PER_TILE)],
                        out_hbm.at[pl.ds(cid * N_PAD + r0, ROWS_PER_TILE)])

    return deg_kernel


def _make_agg_kernel():
    mesh = plsc.VectorSubcoreMesh(core_axis_name="c", subcore_axis_name="s",
                                   num_cores=NC, num_subcores=NS)

    @functools.partial(
        pl.kernel,
        out_type=jax.ShapeDtypeStruct((NC * N_PAD, F_AGG), jnp.float32),
        mesh=mesh,
        scratch_types=[
            pltpu.VMEM((CPW, CHUNK), jnp.int32),
            pltpu.VMEM((CPW, CHUNK), jnp.int32),
            [pltpu.VMEM((CHUNK, F_AGG), jnp.float32) for _ in range(_NBUF)],
            pltpu.VMEM_SHARED((N_PAD, F_AGG), jnp.float32),
            [pltpu.SemaphoreType.DMA for _ in range(_NBUF)],
            [pltpu.SemaphoreType.DMA for _ in range(_NBUF)],
        ],
        compiler_params=pltpu.CompilerParams(use_tc_tiling_on_sc=False),
    )
    def agg_kernel(val_hbm, srcs_hbm, dsts_hbm, zeros_hbm, out_hbm,
                   src_v, dst_v, rows, acc, gsem, ssem):
        cid = lax.axis_index("c")
        sid = lax.axis_index("s")
        r0 = sid * ROWS_PER_TILE
        pltpu.sync_copy(zeros_hbm.at[pl.ds(r0, ROWS_PER_TILE)],
                        acc.at[pl.ds(r0, ROWS_PER_TILE)])
        plsc.subcore_barrier()
        base_chunk = (cid * NS + sid) * CPW
        pltpu.sync_copy(srcs_hbm.at[pl.ds(base_chunk, CPW)], src_v)
        pltpu.sync_copy(dsts_hbm.at[pl.ds(base_chunk, CPW)], dst_v)

        for b in range(_NBUF):
            pltpu.async_copy(val_hbm.at[src_v.at[b]], rows[b], gsem[b])

        def body(i, carry):
            j0 = i * _NBUF
            for b in range(_NBUF):
                j = j0 + b
                pltpu.make_async_copy(val_hbm.at[src_v.at[j]], rows[b],
                                      gsem[b]).wait()
                pltpu.async_copy(rows[b], acc.at[dst_v.at[j]], ssem[b],
                                 add=True)
            for b in range(_NBUF):
                j = j0 + b

                @pl.when(j + _NBUF < CPW)
                def _(b=b, j=j):
                    pltpu.make_async_copy(rows[b], acc.at[dst_v.at[j]],
                                          ssem[b]).wait()
                    pltpu.async_copy(val_hbm.at[src_v.at[j + _NBUF]], rows[b],
                                     gsem[b])
            return carry

        lax.fori_loop(0, CPW // _NBUF, body, 0)
        for b in range(_NBUF):
            pltpu.make_async_copy(rows[b], acc.at[dst_v.at[0]], ssem[b]).wait()
        plsc.subcore_barrier()
        pltpu.sync_copy(acc.at[pl.ds(r0, ROWS_PER_TILE)],
                        out_hbm.at[pl.ds(cid * N_PAD + r0, ROWS_PER_TILE)])

    return agg_kernel


_SC_CACHE = {}


def _deg_call(*args):
    if 'deg' not in _SC_CACHE:
        _SC_CACHE['deg'] = _make_deg_kernel()
    return _SC_CACHE['deg'](*args)


def _agg_call(*args):
    if 'agg' not in _SC_CACHE:
        _SC_CACHE['agg'] = _make_agg_kernel()
    return _SC_CACHE['agg'](*args)


def _enc0_body(x_ref, e0bw, e0sw, e0g, e0b, a0_ref):
    def tile(i, carry):
        xt = x_ref[pl.ds(i * _TILE, _TILE), :]
        a0_ref[pl.ds(i * _TILE, _TILE), :] = _kan_tile(xt, e0bw[...], e0sw)
        return carry
    lax.fori_loop(0, _NTILES, tile, 0)
    a0_ref[...] = _elu(_bn(a0_ref[...], e0g[...], e0b[...]))


def _enc1_body(a0_ref, degp_ref, e1bw, e1sw, e1g, e1b,
               feat_ref, scaled_ref, dinv_ref):
    def tile(i, carry):
        at = a0_ref[pl.ds(i * _TILE, _TILE), :]
        feat_ref[pl.ds(i * _TILE, _TILE), :] = _kan_tile(at, e1bw[...], e1sw)
        return carry
    lax.fori_loop(0, _NTILES, tile, 0)
    feat = _elu(_bn(feat_ref[...], e1g[...], e1b[...]))
    feat_ref[...] = feat
    deg = (degp_ref[pl.ds(0, N_NODES), 0:1]
           + degp_ref[pl.ds(N_PAD, N_NODES), 0:1] + 1.0)
    dinv = lax.rsqrt(deg)
    dinv_ref[...] = dinv
    scaled_ref[pl.ds(0, N_NODES), :] = feat * dinv
    scaled_ref[pl.ds(N_NODES, N_PAD - N_NODES), :] = jnp.zeros(
        (N_PAD - N_NODES, F_AGG), jnp.float32)


def _mid_body(aggp_ref, scaled_ref, dinv_ref, cw, cb, cg, cbeta, mlw,
              ts_ref):
    agg = (aggp_ref[pl.ds(0, N_NODES), :] + aggp_ref[pl.ds(N_PAD, N_NODES), :]
           + scaled_ref[pl.ds(0, N_NODES), :])
    dinv = dinv_ref[...]
    pre = jnp.dot(dinv * agg, cw[...],
                  preferred_element_type=jnp.float32) + cb[...]
    h = jnp.maximum(_bn(pre, cg[...], cbeta[...]), 0.0)
    t = jnp.dot(h, mlw[...], preferred_element_type=jnp.float32)
    ts_ref[pl.ds(0, N_NODES), :] = jnp.concatenate(
        [dinv * t, dinv,
         jnp.zeros((N_NODES, F_AGG - 2 * CONV_HID[1] - 1), jnp.float32)],
        axis=1)
    ts_ref[pl.ds(N_NODES, N_PAD - N_NODES), :] = jnp.zeros(
        (N_PAD - N_NODES, F_AGG), jnp.float32)


def _dec01_body(aggp_ref, ts_ref, feat_ref, mublv,
                d0bw, d0sw, d0g, d0b, d1bw, d1sw, d1g, d1b,
                z_ref, mv_ref, a1_ref, d0_ref):
    tsv = ts_ref[pl.ds(0, N_NODES), :]
    dinv = tsv[:, 2 * CONV_HID[1]:2 * CONV_HID[1] + 1]
    m = ((aggp_ref[pl.ds(0, N_NODES), 0:2 * CONV_HID[1]]
          + aggp_ref[pl.ds(N_PAD, N_NODES), 0:2 * CONV_HID[1]]
          + tsv[:, 0:2 * CONV_HID[1]]) * dinv)
    mv = m + mublv[...]
    mv_ref[...] = mv
    z_ref[...] = jnp.concatenate([feat_ref[...], mv[:, 0:CONV_HID[1]]], axis=1)

    def tile0(i, carry):
        zt = z_ref[pl.ds(i * _TILE, _TILE), :]
        d0_ref[pl.ds(i * _TILE, _TILE), :] = _kan_tile(zt, d0bw[...], d0sw)
        return carry
    lax.fori_loop(0, _NTILES, tile0, 0)
    d0_ref[...] = _elu(_bn(d0_ref[...], d0g[...], d0b[...]))

    def tile1(i, carry):
        at = d0_ref[pl.ds(i * _TILE, _TILE), :]
        a1_ref[pl.ds(i * _TILE, _TILE), :] = _kan_tile(at, d1bw[...], d1sw)
        return carry
    lax.fori_loop(0, _NTILES, tile1, 0)
    a1_ref[...] = _elu(_bn(a1_ref[...], d1g[...], d1b[...]))


def _decout_body(a1_ref, z_ref, dobw, dosw, clus_ref, defeat_ref, q_ref):
    def tile(i, carry):
        at = a1_ref[pl.ds(i * _TILE, _TILE), :]
        defeat_ref[pl.ds(i * _TILE, _TILE), :] = _kan_tile(at, dobw[...], dosw)
        return carry
    lax.fori_loop(0, _NTILES, tile, 0)

    z = z_ref[...]
    clus = clus_ref[...]
    zz = jnp.sum(z * z, axis=1, keepdims=True)
    cc = jnp.sum(clus * clus, axis=1, keepdims=True).T
    zc = jnp.dot(z, clus.T, preferred_element_type=jnp.float32)
    d2 = zz - 2.0 * zc + cc
    t = 1.0 / (1.0 + d2 * (1.0 / ALPHA))
    q = jnp.exp(((ALPHA + 1.0) / 2.0) * jnp.log(t))
    q_ref[...] = q / jnp.sum(q, axis=1, keepdims=True)


def kernel(x, adj, params):
    f32 = jnp.float32
    src = adj[0].astype(jnp.int32)
    dst = adj[1].astype(jnp.int32)
    npad = E_PAD - N_EDGES
    srcs = jnp.concatenate(
        [src, jnp.full((npad,), N_NODES, jnp.int32)]).reshape(NW * CPW, CHUNK)
    dsts = jnp.concatenate(
        [dst, jnp.full((npad,), N_NODES, jnp.int32)]).reshape(NW * CPW, CHUNK)

    zeros8 = jnp.zeros((N_PAD, 8), f32)
    ones8 = jnp.ones((CHUNK, 8), f32)
    zeros32 = jnp.zeros((N_PAD, F_AGG), f32)

    p = params
    e0, e1, d0, d1 = p['enc0'], p['enc1'], p['dec0'], p['dec1']

    def tw(blk):
        return jnp.transpose(blk['spline_w'], (2, 1, 0))

    def row(v):
        return v.reshape(1, -1)

    degp = _deg_call(dsts, zeros8, ones8)

    a0 = pl.pallas_call(
        _enc0_body,
        out_shape=jax.ShapeDtypeStruct((N_NODES, ENC_HID[0]), f32),
    )(x, e0['base_w'].T, tw(e0), row(e0['bn_g']), row(e0['bn_b']))

    feat, scaled, dinv = pl.pallas_call(
        _enc1_body,
        out_shape=[
            jax.ShapeDtypeStruct((N_NODES, ENC_HID[1]), f32),
            jax.ShapeDtypeStruct((N_PAD, F_AGG), f32),
            jax.ShapeDtypeStruct((N_NODES, 1), f32),
        ],
    )(a0, degp, e1['base_w'].T, tw(e1), row(e1['bn_g']), row(e1['bn_b']))

    aggp1 = _agg_call(scaled, srcs, dsts, zeros32)

    mlw = jnp.concatenate([p['mu_w'].T, p['lv_w'].T], axis=1)
    ts = pl.pallas_call(
        _mid_body,
        out_shape=jax.ShapeDtypeStruct((N_PAD, F_AGG), f32),
    )(aggp1, scaled, dinv, p['conv_w'].T, row(p['conv_b']),
      row(p['conv_bn_g']), row(p['conv_bn_b']), mlw)

    aggp2 = _agg_call(ts, srcs, dsts, zeros32)

    mublv = row(jnp.concatenate([p['mu_b'], p['lv_b']]))
    z, mv, a1 = pl.pallas_call(
        _dec01_body,
        out_shape=[
            jax.ShapeDtypeStruct((N_NODES, ZDIM), f32),
            jax.ShapeDtypeStruct((N_NODES, 2 * CONV_HID[1]), f32),
            jax.ShapeDtypeStruct((N_NODES, DEC_HID[1]), f32),
        ],
        scratch_shapes=[pltpu.VMEM((N_NODES, DEC_HID[0]), f32)],
    )(aggp2, ts, feat, mublv,
      d0['base_w'].T, tw(d0), row(d0['bn_g']), row(d0['bn_b']),
      d1['base_w'].T, tw(d1), row(d1['bn_g']), row(d1['bn_b']))

    do = p['dec_out']
    de_feat, q = pl.pallas_call(
        _decout_body,
        out_shape=[
            jax.ShapeDtypeStruct((N_NODES, INPUT_DIM), f32),
            jax.ShapeDtypeStruct((N_NODES, DEC_CLUSTER_N), f32),
        ],
    )(a1, z, do['base_w'].T, tw(do), p['cluster'])

    mu = mv[:, 0:CONV_HID[1]]
    logvar = mv[:, CONV_HID[1]:2 * CONV_HID[1]]
    return (z, mu, logvar, de_feat, q, feat, mu)
